# Initial kernel scaffold; baseline (speedup 1.0000x reference)
import jax
import jax.numpy as jnp
from jax.experimental import pallas as pl


def kernel(ui, uv, ai, av, y, a_table, u_table, fc_w, fc_b):
    raise NotImplementedError("write your pallas kernel here")


# speedup vs baseline: 13.8578x; 13.8578x over previous
import functools

import jax
import jax.numpy as jnp
from jax import lax
from jax.experimental import pallas as pl
from jax.experimental.pallas import tpu as pltpu
from jax.experimental.pallas import tpu_sc as plsc

B = 4096
DAY = 7
AF = 26
UF = 26
EMB = 16
NA = B * DAY * AF
NU = B * UF
AFT = DAY * AF
NC, NS = 2, 16
NW = NC * NS
A_PER_W = NA // NW
U_PER_W = NU // NW
A_CHUNK = 2912
A_STEPS = A_PER_W // A_CHUNK


def _sc_gather(a_table, u_table, ai_flat, ui_flat):
    mesh = plsc.VectorSubcoreMesh(core_axis_name="c", subcore_axis_name="s")

    @functools.partial(
        pl.kernel,
        mesh=mesh,
        compiler_params=pltpu.CompilerParams(use_tc_tiling_on_sc=False),
        out_type=(
            jax.ShapeDtypeStruct((NA, EMB), jnp.float32),
            jax.ShapeDtypeStruct((NU, EMB), jnp.float32),
        ),
        scratch_types=[
            pltpu.VMEM((A_CHUNK,), jnp.int32),
            pltpu.VMEM((A_CHUNK, EMB), jnp.float32),
            pltpu.VMEM((U_PER_W,), jnp.int32),
            pltpu.VMEM((U_PER_W, EMB), jnp.float32),
            pltpu.SemaphoreType.DMA,
        ],
    )
    def gather_kernel(a_tab_hbm, u_tab_hbm, ai_hbm, ui_hbm, a_out, u_out,
                      a_idx_v, a_rows_v, u_idx_v, u_rows_v, sem):
        wid = lax.axis_index("s") * NC + lax.axis_index("c")

        a_base = wid * A_PER_W

        @pl.loop(0, A_STEPS)
        def _(cstep):
            base = a_base + cstep * A_CHUNK
            pltpu.sync_copy(ai_hbm.at[pl.ds(base, A_CHUNK)], a_idx_v)
            pltpu.async_copy(a_tab_hbm.at[a_idx_v], a_rows_v, sem).wait()
            pltpu.sync_copy(a_rows_v, a_out.at[pl.ds(base, A_CHUNK)])

        u_base = wid * U_PER_W
        pltpu.sync_copy(ui_hbm.at[pl.ds(u_base, U_PER_W)], u_idx_v)
        pltpu.async_copy(u_tab_hbm.at[u_idx_v], u_rows_v, sem).wait()
        pltpu.sync_copy(u_rows_v, u_out.at[pl.ds(u_base, U_PER_W)])

    return gather_kernel(a_table, u_table, ai_flat, ui_flat)


BB = 512
ADIM = AFT * EMB
UDIM = UF * EMB


def _tc_reduce_kernel(rows_a_ref, av_ref, rows_u_ref, uv_ref, y_ref,
                      fcw_ref, fcb_ref, ypred_ref, loss_ref,
                      wa_ref, wu_ref):
    i = pl.program_id(0)

    @pl.when(i == 0)
    def _():
        r_a = lax.broadcasted_iota(jnp.int32, (ADIM, AFT), 0)
        c_a = lax.broadcasted_iota(jnp.int32, (ADIM, AFT), 1)
        wa_ref[...] = jnp.where(r_a // EMB == c_a, fcw_ref[:ADIM, :], 0.0)
        r_u = lax.broadcasted_iota(jnp.int32, (UDIM, UF), 0)
        c_u = lax.broadcasted_iota(jnp.int32, (UDIM, UF), 1)
        wu_ref[...] = jnp.where(r_u // EMB == c_u, fcw_ref[ADIM:, :], 0.0)
        loss_ref[...] = jnp.zeros_like(loss_ref)

    dots_a = jnp.dot(rows_a_ref[...], wa_ref[...],
                     preferred_element_type=jnp.float32)
    dots_u = jnp.dot(rows_u_ref[...], wu_ref[...],
                     preferred_element_type=jnp.float32)
    yp = (jnp.sum(dots_a * av_ref[...], axis=1, keepdims=True)
          + jnp.sum(dots_u * uv_ref[...], axis=1, keepdims=True)
          + fcb_ref[0, 0])
    ypred_ref[...] = yp
    loss_ref[...] = loss_ref[...] + jnp.sum((yp - y_ref[...]) ** 2) / B


def _tc_reduce(rows_a, av2, rows_u, uv, y, fc_w, fc_b):
    grid = (B // BB,)
    return pl.pallas_call(
        _tc_reduce_kernel,
        grid=grid,
        in_specs=[
            pl.BlockSpec((BB, ADIM), lambda i: (i, 0)),
            pl.BlockSpec((BB, AFT), lambda i: (i, 0)),
            pl.BlockSpec((BB, UDIM), lambda i: (i, 0)),
            pl.BlockSpec((BB, UF), lambda i: (i, 0)),
            pl.BlockSpec((BB, 1), lambda i: (i, 0)),
            pl.BlockSpec((ADIM + UDIM, 1), lambda i: (0, 0)),
            pl.BlockSpec((1, 1), lambda i: (0, 0)),
        ],
        out_specs=[
            pl.BlockSpec((BB, 1), lambda i: (i, 0)),
            pl.BlockSpec((1, 1), lambda i: (0, 0)),
        ],
        out_shape=[
            jax.ShapeDtypeStruct((B, 1), jnp.float32),
            jax.ShapeDtypeStruct((1, 1), jnp.float32),
        ],
        scratch_shapes=[
            pltpu.VMEM((ADIM, AFT), jnp.float32),
            pltpu.VMEM((UDIM, UF), jnp.float32),
        ],
    )(rows_a, av2, rows_u, uv, y, fc_w, fc_b)


def kernel(ui, uv, ai, av, y, a_table, u_table, fc_w, fc_b):
    ai_flat = ai.reshape(-1).astype(jnp.int32)
    ui_flat = ui.reshape(-1).astype(jnp.int32)
    rows_a, rows_u = _sc_gather(a_table, u_table, ai_flat, ui_flat)
    y_pred, loss = _tc_reduce(
        rows_a.reshape(B, ADIM),
        av.reshape(B, AFT),
        rows_u.reshape(B, UDIM),
        uv.reshape(B, UF),
        y,
        fc_w.reshape(ADIM + UDIM, 1),
        fc_b.reshape(1, 1),
    )
    return loss.reshape(()), y_pred


# speedup vs baseline: 16.4115x; 1.1843x over previous
import functools

import jax
import jax.numpy as jnp
from jax import lax
from jax.experimental import pallas as pl
from jax.experimental.pallas import tpu as pltpu
from jax.experimental.pallas import tpu_sc as plsc

B = 4096
DAY = 7
AF = 26
UF = 26
EMB = 16
AFT = DAY * AF
NA = B * AFT
NU = B * UF
NC, NS = 2, 16
NW = NC * NS
A_PER_W = NA // NW
U_PER_W = NU // NW
A_CHUNK = 2912
A_STEPS = A_PER_W // A_CHUNK
A_ROWS = 1000000
U_ROWS = 100000


A_CBLK = 8192
A_TSTEPS = (A_ROWS + A_CBLK - 1) // A_CBLK
A_LBLK = A_CBLK // 8


def _fold_rows(xt, nrows):
    y3 = xt.reshape(nrows, 8, EMB)
    return jnp.concatenate([y3[:, m, :] for m in range(8)], axis=1)


def _lin_a_kernel(at_ref, out_ref):
    x = at_ref[...]
    out_ref[...] = _fold_rows(x.T, A_LBLK)


def _lin_u_kernel(ut_ref, out_ref):
    x = ut_ref[...]
    out_ref[...] = _fold_rows(x.T, A_LBLK)


def _linearize(a_table, u_table):
    a_lin = pl.pallas_call(
        _lin_a_kernel,
        grid=(A_TSTEPS,),
        in_specs=[pl.BlockSpec((16, A_CBLK), lambda i: (0, i))],
        out_specs=pl.BlockSpec((A_LBLK, 128), lambda i: (i, 0)),
        out_shape=jax.ShapeDtypeStruct((A_ROWS // 8, 128), jnp.float32),
    )(a_table.T)
    u_lin = pl.pallas_call(
        _lin_u_kernel,
        grid=((U_ROWS + A_CBLK - 1) // A_CBLK,),
        in_specs=[pl.BlockSpec((16, A_CBLK), lambda i: (0, i))],
        out_specs=pl.BlockSpec((A_LBLK, 128), lambda i: (i, 0)),
        out_shape=jax.ShapeDtypeStruct((U_ROWS // 8, 128), jnp.float32),
    )(u_table.T)
    return a_lin, u_lin


def _sc_gather(a_tab, u_tab, ai_flat, ui_flat):
    mesh = plsc.VectorSubcoreMesh(core_axis_name="c", subcore_axis_name="s")

    @functools.partial(
        pl.kernel,
        mesh=mesh,
        compiler_params=pltpu.CompilerParams(use_tc_tiling_on_sc=False),
        out_type=(
            jax.ShapeDtypeStruct((NA, EMB), jnp.float32),
            jax.ShapeDtypeStruct((NU, EMB), jnp.float32),
        ),
        scratch_types=[
            pltpu.VMEM((A_CHUNK,), jnp.int32),
            pltpu.VMEM((A_CHUNK, EMB), jnp.float32),
            pltpu.VMEM((U_PER_W,), jnp.int32),
            pltpu.VMEM((U_PER_W, EMB), jnp.float32),
            pltpu.SemaphoreType.DMA,
        ],
    )
    def gather_kernel(a_tab_hbm, u_tab_hbm, ai_hbm, ui_hbm, a_out, u_out,
                      a_idx_v, a_rows_v, u_idx_v, u_rows_v, sem):
        wid = lax.axis_index("s") * NC + lax.axis_index("c")

        a_base = wid * A_PER_W

        @pl.loop(0, A_STEPS)
        def _(cstep):
            base = a_base + cstep * A_CHUNK
            pltpu.sync_copy(ai_hbm.at[pl.ds(base, A_CHUNK)], a_idx_v)
            pltpu.async_copy(a_tab_hbm.at[a_idx_v], a_rows_v, sem).wait()
            pltpu.sync_copy(a_rows_v, a_out.at[pl.ds(base, A_CHUNK)])

        u_base = wid * U_PER_W
        pltpu.sync_copy(ui_hbm.at[pl.ds(u_base, U_PER_W)], u_idx_v)
        pltpu.async_copy(u_tab_hbm.at[u_idx_v], u_rows_v, sem).wait()
        pltpu.sync_copy(u_rows_v, u_out.at[pl.ds(u_base, U_PER_W)])

    return gather_kernel(a_tab, u_tab, ai_flat, ui_flat)


BB = 512
ADIM = AFT * EMB
UDIM = UF * EMB


def _tc_reduce_kernel(rows_a_ref, av_ref, rows_u_ref, uv_ref, y_ref,
                      fcw_ref, fcb_ref, ypred_ref, loss_ref,
                      wa_ref, wu_ref):
    i = pl.program_id(0)

    @pl.when(i == 0)
    def _():
        r_a = lax.broadcasted_iota(jnp.int32, (ADIM, AFT), 0)
        c_a = lax.broadcasted_iota(jnp.int32, (ADIM, AFT), 1)
        wa_ref[...] = jnp.where(r_a // EMB == c_a, fcw_ref[:ADIM, :], 0.0)
        r_u = lax.broadcasted_iota(jnp.int32, (UDIM, UF), 0)
        c_u = lax.broadcasted_iota(jnp.int32, (UDIM, UF), 1)
        wu_ref[...] = jnp.where(r_u // EMB == c_u, fcw_ref[ADIM:, :], 0.0)
        loss_ref[...] = jnp.zeros_like(loss_ref)

    dots_a = jnp.dot(rows_a_ref[...], wa_ref[...],
                     preferred_element_type=jnp.float32)
    dots_u = jnp.dot(rows_u_ref[...], wu_ref[...],
                     preferred_element_type=jnp.float32)
    yp = (jnp.sum(dots_a * av_ref[...], axis=1, keepdims=True)
          + jnp.sum(dots_u * uv_ref[...], axis=1, keepdims=True)
          + fcb_ref[0, 0])
    ypred_ref[...] = yp
    loss_ref[...] = loss_ref[...] + jnp.sum((yp - y_ref[...]) ** 2) / B


def _tc_reduce(rows_a, av2, rows_u, uv, y, fc_w, fc_b):
    return pl.pallas_call(
        _tc_reduce_kernel,
        grid=(B // BB,),
        in_specs=[
            pl.BlockSpec((BB, ADIM), lambda i: (i, 0)),
            pl.BlockSpec((BB, AFT), lambda i: (i, 0)),
            pl.BlockSpec((BB, UDIM), lambda i: (i, 0)),
            pl.BlockSpec((BB, UF), lambda i: (i, 0)),
            pl.BlockSpec((BB, 1), lambda i: (i, 0)),
            pl.BlockSpec((ADIM + UDIM, 1), lambda i: (0, 0)),
            pl.BlockSpec((1, 1), lambda i: (0, 0)),
        ],
        out_specs=[
            pl.BlockSpec((BB, 1), lambda i: (i, 0)),
            pl.BlockSpec((1, 1), lambda i: (0, 0)),
        ],
        out_shape=[
            jax.ShapeDtypeStruct((B, 1), jnp.float32),
            jax.ShapeDtypeStruct((1, 1), jnp.float32),
        ],
        scratch_shapes=[
            pltpu.VMEM((ADIM, AFT), jnp.float32),
            pltpu.VMEM((UDIM, UF), jnp.float32),
        ],
    )(rows_a, av2, rows_u, uv, y, fc_w, fc_b)


def kernel(ui, uv, ai, av, y, a_table, u_table, fc_w, fc_b):
    a_lin, u_lin = _linearize(a_table, u_table)
    ai_flat = ai.reshape(-1).astype(jnp.int32)
    ui_flat = ui.reshape(-1).astype(jnp.int32)
    rows_a, rows_u = _sc_gather(
        a_lin.reshape(A_ROWS, EMB), u_lin.reshape(U_ROWS, EMB),
        ai_flat, ui_flat)
    y_pred, loss = _tc_reduce(
        rows_a.reshape(B, ADIM),
        av.reshape(B, AFT),
        rows_u.reshape(B, UDIM),
        uv.reshape(B, UF),
        y,
        fc_w.reshape(ADIM + UDIM, 1),
        fc_b.reshape(1, 1),
    )
    return loss.reshape(()), y_pred
